# Initial kernel scaffold; baseline (speedup 1.0000x reference)
import jax
import jax.numpy as jnp
from jax.experimental import pallas as pl


def kernel(x, As, ycl, iteration, node_features, Wf, bf, Wp, bp, Wo, bo, enc0_Wru, enc0_bru, enc0_Wc, enc0_bc, enc1_Wru, enc1_bru, enc1_Wc, enc1_bc, dec0_Wru, dec0_bru, dec0_Wc, dec0_bc, dec1_Wru, dec1_bru, dec1_Wc, dec1_bc, Wproj, bproj):
    raise NotImplementedError("write your pallas kernel here")


# speedup vs baseline: 1.0831x; 1.0831x over previous
import functools

import jax
import jax.numpy as jnp
from jax.experimental import pallas as pl
from jax.experimental.pallas import tpu as pltpu

_N = 512
_H = 64
_B = 4
_P = 12
_Q = 12
_RB = 64


def _gts_kernel(nfT, xin, Wf, bf, Wp, bp, WoT, bo,
                W0ru, b0ru, W0c, b0c, W1ru, b1ru, W1c, b1c,
                W2ru, b2ru, W2c, b2c, W3ru, b3ru, W3c, b3c,
                Wproj, bproj, out_ref, A_scr):
    N, H, B = _N, _H, _B

    feat = jnp.maximum(nfT[:] @ Wf[:] + bf[:], 0.0)
    Lp = feat @ Wp[0:H, :] + bp[:]
    Rp = feat @ Wp[H:2 * H, :]
    wv = (WoT[0:1, :] - WoT[1:2, :])[None]
    c0 = bo[0, 0] - bo[0, 1]

    for i in range(N // _RB):
        Lb = Lp[i * _RB:(i + 1) * _RB, :]
        t3 = jnp.maximum(Lb[:, None, :] + Rp[None, :, :], 0.0)
        a = jnp.sum(t3 * wv, axis=2)
        A_scr[i * _RB:(i + 1) * _RB, :] = jax.nn.sigmoid(2.0 * (a + c0))
    Araw = A_scr[:]
    Ah = Araw / (jnp.sum(Araw, axis=1, keepdims=True) + 1e-8)

    def gconv(xt, h, W, bias, ind):
        F = ind + H
        parts = []
        for b in range(B):
            parts.append(xt[:, b * ind:(b + 1) * ind])
            parts.append(h[:, b * H:(b + 1) * H])
        cat = jnp.concatenate(parts, axis=1)
        x1 = Ah @ cat
        x2 = Ah @ x1
        outs = []
        for b in range(B):
            c3 = jnp.concatenate([cat[:, b * F:(b + 1) * F],
                                  x1[:, b * F:(b + 1) * F],
                                  x2[:, b * F:(b + 1) * F]], axis=1)
            outs.append(c3 @ W[:] + bias[:])
        return jnp.concatenate(outs, axis=1)

    def cell(xt, h, Wru, bru, Wc, bc, ind):
        ru = jax.nn.sigmoid(gconv(xt, h, Wru, bru, ind))
        r = jnp.concatenate(
            [ru[:, b * 2 * H:b * 2 * H + H] for b in range(B)], axis=1)
        u = jnp.concatenate(
            [ru[:, b * 2 * H + H:(b + 1) * 2 * H] for b in range(B)], axis=1)
        c = jnp.tanh(gconv(xt, r * h, Wc, bc, ind))
        return u * h + (1.0 - u) * c

    z = jnp.zeros((N, B * H), jnp.float32)
    h0, h1 = z, z
    for t in range(_P):
        xt = xin[:, t * B:(t + 1) * B]
        h0 = cell(xt, h0, W0ru, b0ru, W0c, b0c, 1)
        h1 = cell(h0, h1, W1ru, b1ru, W1c, b1c, H)

    g0, g1 = h0, h1
    xq = jnp.zeros((N, B), jnp.float32)
    for q in range(_Q):
        g0 = cell(xq, g0, W2ru, b2ru, W2c, b2c, 1)
        g1 = cell(g0, g1, W3ru, b3ru, W3c, b3c, H)
        outs = [g1[:, b * H:(b + 1) * H] @ Wproj[:] + bproj[:] for b in range(B)]
        xq = jnp.concatenate(outs, axis=1)
        out_ref[:, q * B:(q + 1) * B] = xq


@jax.jit
def _run(x, node_features, Wf, bf, Wp, bp, Wo, bo,
         enc0_Wru, enc0_bru, enc0_Wc, enc0_bc,
         enc1_Wru, enc1_bru, enc1_Wc, enc1_bc,
         dec0_Wru, dec0_bru, dec0_Wc, dec0_bc,
         dec1_Wru, dec1_bru, dec1_Wc, dec1_bc,
         Wproj, bproj):
    nfT = node_features.T
    xin = jnp.transpose(x[..., 0], (2, 1, 0)).reshape(_N, _P * _B)
    r2 = lambda v: v.reshape(1, -1)

    out = pl.pallas_call(
        _gts_kernel,
        out_shape=jax.ShapeDtypeStruct((_N, _Q * _B), jnp.float32),
        scratch_shapes=[pltpu.VMEM((_N, _N), jnp.float32)],
    )(nfT, xin, Wf, r2(bf), Wp, r2(bp), Wo.T, r2(bo),
      enc0_Wru, r2(enc0_bru), enc0_Wc, r2(enc0_bc),
      enc1_Wru, r2(enc1_bru), enc1_Wc, r2(enc1_bc),
      dec0_Wru, r2(dec0_bru), dec0_Wc, r2(dec0_bc),
      dec1_Wru, r2(dec1_bru), dec1_Wc, r2(dec1_bc),
      Wproj, r2(bproj))
    return jnp.transpose(out.reshape(_N, _Q, _B), (2, 1, 0))


def kernel(x, As, ycl, iteration, node_features, Wf, bf, Wp, bp, Wo, bo,
           enc0_Wru, enc0_bru, enc0_Wc, enc0_bc,
           enc1_Wru, enc1_bru, enc1_Wc, enc1_bc,
           dec0_Wru, dec0_bru, dec0_Wc, dec0_bc,
           dec1_Wru, dec1_bru, dec1_Wc, dec1_bc,
           Wproj, bproj):
    return _run(x, node_features, Wf, bf, Wp, bp, Wo, bo,
                enc0_Wru, enc0_bru, enc0_Wc, enc0_bc,
                enc1_Wru, enc1_bru, enc1_Wc, enc1_bc,
                dec0_Wru, dec0_bru, dec0_Wc, dec0_bc,
                dec1_Wru, dec1_bru, dec1_Wc, dec1_bc,
                Wproj, bproj)


# speedup vs baseline: 1.1836x; 1.0928x over previous
import functools

import jax
import jax.numpy as jnp
from jax.experimental import pallas as pl
from jax.experimental.pallas import tpu as pltpu

_N = 512
_H = 64
_B = 4
_P = 12
_Q = 12
_RB = 64


def _gts_kernel(nfT, xin, Wf, bf, Wp, bp, WoT, bo,
                W0ru, b0ru, W0c, b0c, W1ru, b1ru, W1c, b1c,
                W2ru, b2ru, W2c, b2c, W3ru, b3ru, W3c, b3c,
                Wproj, bproj, out_ref, A_scr):
    N, H, B = _N, _H, _B

    feat = jnp.maximum(nfT[:] @ Wf[:] + bf[:], 0.0)
    Lp = feat @ Wp[0:H, :] + bp[:]
    Rp = feat @ Wp[H:2 * H, :]
    wv = (WoT[0:1, :] - WoT[1:2, :])[None]
    c0 = bo[0, 0] - bo[0, 1]

    for i in range(N // _RB):
        Lb = Lp[i * _RB:(i + 1) * _RB, :]
        t3 = jnp.maximum(Lb[:, None, :] + Rp[None, :, :], 0.0)
        a = jnp.sum(t3 * wv, axis=2)
        A_scr[i * _RB:(i + 1) * _RB, :] = jax.nn.sigmoid(2.0 * (a + c0))
    Araw = A_scr[:]
    Ah = Araw / (jnp.sum(Araw, axis=1, keepdims=True) + 1e-8)

    bf = jnp.bfloat16
    Ab = Ah.astype(bf)
    Wbf = [w[:].astype(bf) for w in (W0ru, W0c, W1ru, W1c,
                                     W2ru, W2c, W3ru, W3c)]
    W0ru_b, W0c_b, W1ru_b, W1c_b, W2ru_b, W2c_b, W3ru_b, W3c_b = Wbf

    def hop2(xb):
        x1 = jnp.dot(Ab, xb, preferred_element_type=jnp.float32).astype(bf)
        x2 = jnp.dot(Ab, x1, preferred_element_type=jnp.float32).astype(bf)
        return x1, x2

    def wmm(p0, ph, p1, q1, p2, q2, W, bias, ind):
        outs = []
        for b in range(B):
            sx = slice(b * ind, (b + 1) * ind)
            sh = slice(b * H, (b + 1) * H)
            c3 = jnp.concatenate([p0[:, sx], ph[:, sh], p1[:, sx],
                                  q1[:, sh], p2[:, sx], q2[:, sh]], axis=1)
            outs.append(
                jnp.dot(c3, W, preferred_element_type=jnp.float32) + bias[:])
        return jnp.concatenate(outs, axis=1)

    def cell(xt, h, Wru, bru, Wc, bc, ind):
        xb = xt.astype(bf)
        hb = h.astype(bf)
        p1, p2 = hop2(xb)
        q1, q2 = hop2(hb)
        ru = jax.nn.sigmoid(wmm(xb, hb, p1, q1, p2, q2, Wru, bru, ind))
        r = jnp.concatenate(
            [ru[:, b * 2 * H:b * 2 * H + H] for b in range(B)], axis=1)
        u = jnp.concatenate(
            [ru[:, b * 2 * H + H:(b + 1) * 2 * H] for b in range(B)], axis=1)
        sb = (r * h).astype(bf)
        s1, s2 = hop2(sb)
        c = jnp.tanh(wmm(xb, sb, p1, s1, p2, s2, Wc, bc, ind))
        return u * h + (1.0 - u) * c

    z = jnp.zeros((N, B * H), jnp.float32)
    h0, h1 = z, z
    for t in range(_P):
        xt = xin[:, t * B:(t + 1) * B]
        h0 = cell(xt, h0, W0ru_b, b0ru, W0c_b, b0c, 1)
        h1 = cell(h0, h1, W1ru_b, b1ru, W1c_b, b1c, H)

    g0, g1 = h0, h1
    xq = jnp.zeros((N, B), jnp.float32)
    for q in range(_Q):
        g0 = cell(xq, g0, W2ru_b, b2ru, W2c_b, b2c, 1)
        g1 = cell(g0, g1, W3ru_b, b3ru, W3c_b, b3c, H)
        outs = [g1[:, b * H:(b + 1) * H] @ Wproj[:] + bproj[:] for b in range(B)]
        xq = jnp.concatenate(outs, axis=1)
        out_ref[:, q * B:(q + 1) * B] = xq


@jax.jit
def _run(x, node_features, Wf, bf, Wp, bp, Wo, bo,
         enc0_Wru, enc0_bru, enc0_Wc, enc0_bc,
         enc1_Wru, enc1_bru, enc1_Wc, enc1_bc,
         dec0_Wru, dec0_bru, dec0_Wc, dec0_bc,
         dec1_Wru, dec1_bru, dec1_Wc, dec1_bc,
         Wproj, bproj):
    nfT = node_features.T
    xin = jnp.transpose(x[..., 0], (2, 1, 0)).reshape(_N, _P * _B)
    r2 = lambda v: v.reshape(1, -1)

    out = pl.pallas_call(
        _gts_kernel,
        out_shape=jax.ShapeDtypeStruct((_N, _Q * _B), jnp.float32),
        scratch_shapes=[pltpu.VMEM((_N, _N), jnp.float32)],
    )(nfT, xin, Wf, r2(bf), Wp, r2(bp), Wo.T, r2(bo),
      enc0_Wru, r2(enc0_bru), enc0_Wc, r2(enc0_bc),
      enc1_Wru, r2(enc1_bru), enc1_Wc, r2(enc1_bc),
      dec0_Wru, r2(dec0_bru), dec0_Wc, r2(dec0_bc),
      dec1_Wru, r2(dec1_bru), dec1_Wc, r2(dec1_bc),
      Wproj, r2(bproj))
    return jnp.transpose(out.reshape(_N, _Q, _B), (2, 1, 0))


def kernel(x, As, ycl, iteration, node_features, Wf, bf, Wp, bp, Wo, bo,
           enc0_Wru, enc0_bru, enc0_Wc, enc0_bc,
           enc1_Wru, enc1_bru, enc1_Wc, enc1_bc,
           dec0_Wru, dec0_bru, dec0_Wc, dec0_bc,
           dec1_Wru, dec1_bru, dec1_Wc, dec1_bc,
           Wproj, bproj):
    return _run(x, node_features, Wf, bf, Wp, bp, Wo, bo,
                enc0_Wru, enc0_bru, enc0_Wc, enc0_bc,
                enc1_Wru, enc1_bru, enc1_Wc, enc1_bc,
                dec0_Wru, dec0_bru, dec0_Wc, dec0_bc,
                dec1_Wru, dec1_bru, dec1_Wc, dec1_bc,
                Wproj, bproj)


# speedup vs baseline: 1.1866x; 1.0025x over previous
import functools

import jax
import jax.numpy as jnp
from jax.experimental import pallas as pl
from jax.experimental.pallas import tpu as pltpu

_N = 512
_H = 64
_B = 4
_P = 12
_Q = 12
_RB = 64


def _gts_kernel(nfT, xin, Wf, bf, Wp, bp, WoT, bo,
                W0ru, b0ru, W0c, b0c, W1ru, b1ru, W1c, b1c,
                W2ru, b2ru, W2c, b2c, W3ru, b3ru, W3c, b3c,
                Wproj, bproj, out_ref, A_scr):
    N, H, B = _N, _H, _B

    learner_scope = jax.named_scope("learner")
    learner_scope.__enter__()
    feat = jnp.maximum(nfT[:] @ Wf[:] + bf[:], 0.0)
    Lp = feat @ Wp[0:H, :] + bp[:]
    Rp = feat @ Wp[H:2 * H, :]
    wv = (WoT[0:1, :] - WoT[1:2, :])[None]
    c0 = bo[0, 0] - bo[0, 1]

    for i in range(N // _RB):
        Lb = Lp[i * _RB:(i + 1) * _RB, :]
        t3 = jnp.maximum(Lb[:, None, :] + Rp[None, :, :], 0.0)
        a = jnp.sum(t3 * wv, axis=2)
        A_scr[i * _RB:(i + 1) * _RB, :] = jax.nn.sigmoid(2.0 * (a + c0))
    Araw = A_scr[:]
    Ah = Araw / (jnp.sum(Araw, axis=1, keepdims=True) + 1e-8)
    learner_scope.__exit__(None, None, None)

    bf = jnp.bfloat16
    Ab = Ah.astype(bf)
    Wbf = [w[:].astype(bf) for w in (W0ru, W0c, W1ru, W1c,
                                     W2ru, W2c, W3ru, W3c)]
    W0ru_b, W0c_b, W1ru_b, W1c_b, W2ru_b, W2c_b, W3ru_b, W3c_b = Wbf

    def hop2(xb):
        x1 = jnp.dot(Ab, xb, preferred_element_type=jnp.float32).astype(bf)
        x2 = jnp.dot(Ab, x1, preferred_element_type=jnp.float32).astype(bf)
        return x1, x2

    def wmm(p0, ph, p1, q1, p2, q2, W, bias, ind):
        outs = []
        for b in range(B):
            sx = slice(b * ind, (b + 1) * ind)
            sh = slice(b * H, (b + 1) * H)
            c3 = jnp.concatenate([p0[:, sx], ph[:, sh], p1[:, sx],
                                  q1[:, sh], p2[:, sx], q2[:, sh]], axis=1)
            outs.append(
                jnp.dot(c3, W, preferred_element_type=jnp.float32) + bias[:])
        return jnp.concatenate(outs, axis=1)

    def cell(xt, h, Wru, bru, Wc, bc, ind):
        xb = xt.astype(bf)
        hb = h.astype(bf)
        p1, p2 = hop2(xb)
        q1, q2 = hop2(hb)
        ru = jax.nn.sigmoid(wmm(xb, hb, p1, q1, p2, q2, Wru, bru, ind))
        r = jnp.concatenate(
            [ru[:, b * 2 * H:b * 2 * H + H] for b in range(B)], axis=1)
        u = jnp.concatenate(
            [ru[:, b * 2 * H + H:(b + 1) * 2 * H] for b in range(B)], axis=1)
        sb = (r * h).astype(bf)
        s1, s2 = hop2(sb)
        c = jnp.tanh(wmm(xb, sb, p1, s1, p2, s2, Wc, bc, ind))
        return u * h + (1.0 - u) * c

    z = jnp.zeros((N, B * H), jnp.float32)
    h0, h1 = z, z
    for t in range(_P):
        with jax.named_scope(f"enc{t}"):
            xt = xin[:, t * B:(t + 1) * B]
            h0 = cell(xt, h0, W0ru_b, b0ru, W0c_b, b0c, 1)
            h1 = cell(h0, h1, W1ru_b, b1ru, W1c_b, b1c, H)

    g0, g1 = h0, h1
    xq = jnp.zeros((N, B), jnp.float32)
    for q in range(_Q):
        with jax.named_scope(f"dec{q}"):
            g0 = cell(xq, g0, W2ru_b, b2ru, W2c_b, b2c, 1)
            g1 = cell(g0, g1, W3ru_b, b3ru, W3c_b, b3c, H)
            outs = [g1[:, b * H:(b + 1) * H] @ Wproj[:] + bproj[:]
                    for b in range(B)]
            xq = jnp.concatenate(outs, axis=1)
            out_ref[:, q * B:(q + 1) * B] = xq


@jax.jit
def _run(x, node_features, Wf, bf, Wp, bp, Wo, bo,
         enc0_Wru, enc0_bru, enc0_Wc, enc0_bc,
         enc1_Wru, enc1_bru, enc1_Wc, enc1_bc,
         dec0_Wru, dec0_bru, dec0_Wc, dec0_bc,
         dec1_Wru, dec1_bru, dec1_Wc, dec1_bc,
         Wproj, bproj):
    nfT = node_features.T
    xin = jnp.transpose(x[..., 0], (2, 1, 0)).reshape(_N, _P * _B)
    r2 = lambda v: v.reshape(1, -1)

    out = pl.pallas_call(
        _gts_kernel,
        out_shape=jax.ShapeDtypeStruct((_N, _Q * _B), jnp.float32),
        scratch_shapes=[pltpu.VMEM((_N, _N), jnp.float32)],
    )(nfT, xin, Wf, r2(bf), Wp, r2(bp), Wo.T, r2(bo),
      enc0_Wru, r2(enc0_bru), enc0_Wc, r2(enc0_bc),
      enc1_Wru, r2(enc1_bru), enc1_Wc, r2(enc1_bc),
      dec0_Wru, r2(dec0_bru), dec0_Wc, r2(dec0_bc),
      dec1_Wru, r2(dec1_bru), dec1_Wc, r2(dec1_bc),
      Wproj, r2(bproj))
    return jnp.transpose(out.reshape(_N, _Q, _B), (2, 1, 0))


def kernel(x, As, ycl, iteration, node_features, Wf, bf, Wp, bp, Wo, bo,
           enc0_Wru, enc0_bru, enc0_Wc, enc0_bc,
           enc1_Wru, enc1_bru, enc1_Wc, enc1_bc,
           dec0_Wru, dec0_bru, dec0_Wc, dec0_bc,
           dec1_Wru, dec1_bru, dec1_Wc, dec1_bc,
           Wproj, bproj):
    return _run(x, node_features, Wf, bf, Wp, bp, Wo, bo,
                enc0_Wru, enc0_bru, enc0_Wc, enc0_bc,
                enc1_Wru, enc1_bru, enc1_Wc, enc1_bc,
                dec0_Wru, dec0_bru, dec0_Wc, dec0_bc,
                dec1_Wru, dec1_bru, dec1_Wc, dec1_bc,
                Wproj, bproj)


# speedup vs baseline: 1.5349x; 1.2936x over previous
import functools

import jax
import jax.numpy as jnp
from jax.experimental import pallas as pl
from jax.experimental.pallas import tpu as pltpu

_N = 512
_H = 64
_B = 4
_P = 12
_Q = 12
_RB = 64


def _gts_kernel(nf, nfT, WfT, WpRT, xin, Wf, bf, bfT, Wp, bp, WoT, bo,
                W0ru, b0ru, W0c, b0c, W1ru, b1ru, W1c, b1c,
                W2ru, b2ru, W2c, b2c, W3ru, b3ru, W3c, b3c,
                Wproj, bproj, out_ref):
    N, H, B = _N, _H, _B

    feat = jnp.maximum(nfT[:] @ Wf[:] + bf[:], 0.0)
    featT = jnp.maximum(WfT[:] @ nf[:] + bfT[:], 0.0)
    Lp = feat @ Wp[0:H, :] + bp[:]
    RpT = WpRT[:] @ featT
    wv = WoT[0:1, :] - WoT[1:2, :]
    c0 = bo[0, 0] - bo[0, 1]

    acc = jnp.zeros((N, N), jnp.float32)
    for h in range(H):
        acc = acc + wv[0, h] * jnp.maximum(
            Lp[:, h:h + 1] + RpT[h:h + 1, :], 0.0)
    Araw = jax.nn.sigmoid(2.0 * (acc + c0))
    Ah = Araw / (jnp.sum(Araw, axis=1, keepdims=True) + 1e-8)

    bf = jnp.bfloat16
    Ab = Ah.astype(bf)
    Wbf = [w[:].astype(bf) for w in (W0ru, W0c, W1ru, W1c,
                                     W2ru, W2c, W3ru, W3c)]
    W0ru_b, W0c_b, W1ru_b, W1c_b, W2ru_b, W2c_b, W3ru_b, W3c_b = Wbf

    def hop2(xb):
        x1 = jnp.dot(Ab, xb, preferred_element_type=jnp.float32).astype(bf)
        x2 = jnp.dot(Ab, x1, preferred_element_type=jnp.float32).astype(bf)
        return x1, x2

    def wmm(p0, ph, p1, q1, p2, q2, W, bias, ind):
        outs = []
        for b in range(B):
            sx = slice(b * ind, (b + 1) * ind)
            sh = slice(b * H, (b + 1) * H)
            c3 = jnp.concatenate([p0[:, sx], ph[:, sh], p1[:, sx],
                                  q1[:, sh], p2[:, sx], q2[:, sh]], axis=1)
            outs.append(
                jnp.dot(c3, W, preferred_element_type=jnp.float32) + bias[:])
        return jnp.concatenate(outs, axis=1)

    def cell(xt, h, Wru, bru, Wc, bc, ind):
        xb = xt.astype(bf)
        hb = h.astype(bf)
        p1, p2 = hop2(xb)
        q1, q2 = hop2(hb)
        ru = jax.nn.sigmoid(wmm(xb, hb, p1, q1, p2, q2, Wru, bru, ind))
        r = jnp.concatenate(
            [ru[:, b * 2 * H:b * 2 * H + H] for b in range(B)], axis=1)
        u = jnp.concatenate(
            [ru[:, b * 2 * H + H:(b + 1) * 2 * H] for b in range(B)], axis=1)
        sb = (r * h).astype(bf)
        s1, s2 = hop2(sb)
        c = jnp.tanh(wmm(xb, sb, p1, s1, p2, s2, Wc, bc, ind))
        return u * h + (1.0 - u) * c

    z = jnp.zeros((N, B * H), jnp.float32)
    h0, h1 = z, z
    for t in range(_P):
        xt = xin[:, t * B:(t + 1) * B]
        h0 = cell(xt, h0, W0ru_b, b0ru, W0c_b, b0c, 1)
        h1 = cell(h0, h1, W1ru_b, b1ru, W1c_b, b1c, H)

    g0, g1 = h0, h1
    xq = jnp.zeros((N, B), jnp.float32)
    for q in range(_Q):
        g0 = cell(xq, g0, W2ru_b, b2ru, W2c_b, b2c, 1)
        g1 = cell(g0, g1, W3ru_b, b3ru, W3c_b, b3c, H)
        outs = [g1[:, b * H:(b + 1) * H] @ Wproj[:] + bproj[:]
                for b in range(B)]
        xq = jnp.concatenate(outs, axis=1)
        out_ref[:, q * B:(q + 1) * B] = xq


@jax.jit
def _run(x, node_features, Wf, bf, Wp, bp, Wo, bo,
         enc0_Wru, enc0_bru, enc0_Wc, enc0_bc,
         enc1_Wru, enc1_bru, enc1_Wc, enc1_bc,
         dec0_Wru, dec0_bru, dec0_Wc, dec0_bc,
         dec1_Wru, dec1_bru, dec1_Wc, dec1_bc,
         Wproj, bproj):
    nfT = node_features.T
    xin = jnp.transpose(x[..., 0], (2, 1, 0)).reshape(_N, _P * _B)
    r2 = lambda v: v.reshape(1, -1)

    out = pl.pallas_call(
        _gts_kernel,
        out_shape=jax.ShapeDtypeStruct((_N, _Q * _B), jnp.float32),
    )(node_features, nfT, Wf.T, Wp[_H:2 * _H, :].T, xin,
      Wf, r2(bf), bf.reshape(-1, 1), Wp, r2(bp), Wo.T, r2(bo),
      enc0_Wru, r2(enc0_bru), enc0_Wc, r2(enc0_bc),
      enc1_Wru, r2(enc1_bru), enc1_Wc, r2(enc1_bc),
      dec0_Wru, r2(dec0_bru), dec0_Wc, r2(dec0_bc),
      dec1_Wru, r2(dec1_bru), dec1_Wc, r2(dec1_bc),
      Wproj, r2(bproj))
    return jnp.transpose(out.reshape(_N, _Q, _B), (2, 1, 0))


def kernel(x, As, ycl, iteration, node_features, Wf, bf, Wp, bp, Wo, bo,
           enc0_Wru, enc0_bru, enc0_Wc, enc0_bc,
           enc1_Wru, enc1_bru, enc1_Wc, enc1_bc,
           dec0_Wru, dec0_bru, dec0_Wc, dec0_bc,
           dec1_Wru, dec1_bru, dec1_Wc, dec1_bc,
           Wproj, bproj):
    return _run(x, node_features, Wf, bf, Wp, bp, Wo, bo,
                enc0_Wru, enc0_bru, enc0_Wc, enc0_bc,
                enc1_Wru, enc1_bru, enc1_Wc, enc1_bc,
                dec0_Wru, dec0_bru, dec0_Wc, dec0_bc,
                dec1_Wru, dec1_bru, dec1_Wc, dec1_bc,
                Wproj, bproj)


# speedup vs baseline: 1.9128x; 1.2462x over previous
import jax
import jax.numpy as jnp
from jax.experimental import pallas as pl

_N = 512
_H = 64
_B = 4
_P = 12
_Q = 12
_BN = _B * _N
_IP = 8


def _gts_kernel(nf, nfT, Wf, WfT, bf_r, bf_c, WpR, WpLT, bp_c, WoT, bo,
                xin, W0ru, b0ru, W0c, b0c, W1ru, b1ru, W1c, b1c,
                W2ru, b2ru, W2c, b2c, W3ru, b3ru, W3c, b3c,
                Wproj, bproj, out_ref):
    N, H, B, BN = _N, _H, _B, _BN
    bf16 = jnp.bfloat16

    feat = jnp.maximum(nfT[:] @ Wf[:] + bf_r[:], 0.0)
    featT = jnp.maximum(WfT[:] @ nf[:] + bf_c[:], 0.0)
    Rp = feat @ WpR[:]
    LpT = WpLT[:] @ featT + bp_c[:]
    wv = WoT[0:1, :] - WoT[1:2, :]
    c0 = bo[0, 0] - bo[0, 1]

    accT = jnp.zeros((N, N), jnp.float32)
    for h in range(H):
        accT = accT + wv[0, h] * jnp.maximum(
            Rp[:, h:h + 1] + LpT[h:h + 1, :], 0.0)
    AT = jax.nn.sigmoid(2.0 * (accT + c0))
    colsum = jnp.sum(AT, axis=0, keepdims=True)
    AhT = (AT / (colsum + 1e-8)).astype(bf16)

    def hopall(X):
        return jnp.concatenate(
            [jnp.dot(X[:, b * N:(b + 1) * N], AhT,
                     preferred_element_type=jnp.float32).astype(bf16)
             for b in range(B)], axis=1)

    def wmm(c3s, WT, bias_c):
        W = WT[:]
        return jnp.concatenate(
            [jnp.dot(W, c3, preferred_element_type=jnp.float32)
             for c3 in c3s], axis=1) + bias_c[:]

    def cell(xt, h, WruT, bruT, WcT, bcT, ind):
        hb = h.astype(bf16)
        cat1 = jnp.concatenate([xt, hb], axis=0)
        x1 = hopall(cat1)
        x2 = hopall(x1)
        sl = lambda M, b: M[:, b * N:(b + 1) * N]
        ru = jax.nn.sigmoid(wmm(
            [jnp.concatenate([sl(cat1, b), sl(x1, b), sl(x2, b)], axis=0)
             for b in range(B)], WruT, bruT))
        r = ru[0:H, :]
        u = ru[H:2 * H, :]
        s = (r * h).astype(bf16)
        cat2 = jnp.concatenate([xt, s], axis=0)
        s1 = hopall(s)
        x1b = jnp.concatenate([x1[0:ind, :], s1], axis=0)
        s2 = hopall(s1)
        x2b = jnp.concatenate([x2[0:ind, :], s2], axis=0)
        c = jnp.tanh(wmm(
            [jnp.concatenate([sl(cat2, b), sl(x1b, b), sl(x2b, b)], axis=0)
             for b in range(B)], WcT, bcT))
        return u * h + (1.0 - u) * c

    z = jnp.zeros((H, BN), jnp.float32)
    h0, h1 = z, z
    for t in range(_P):
        xt = xin[t * _IP:(t + 1) * _IP, :].astype(bf16)
        h0 = cell(xt, h0, W0ru, b0ru, W0c, b0c, _IP)
        h1 = cell(h0.astype(bf16), h1, W1ru, b1ru, W1c, b1c, H)

    g0, g1 = h0, h1
    zq = jnp.zeros((_IP - 1, BN), bf16)
    xq = jnp.zeros((1, BN), bf16)
    for q in range(_Q):
        xq8 = jnp.concatenate([xq, zq], axis=0)
        g0 = cell(xq8, g0, W2ru, b2ru, W2c, b2c, _IP)
        g1 = cell(g0.astype(bf16), g1, W3ru, b3ru, W3c, b3c, H)
        pr = jnp.sum(g1 * Wproj[:], axis=0, keepdims=True) + bproj[0, 0]
        xq = pr.astype(bf16)
        for b in range(B):
            out_ref[q * B + b:q * B + b + 1, :] = pr[:, b * N:(b + 1) * N]


def _pad_narrow(W):
    F = 1 + _H
    blocks = []
    for k in range(3):
        blk = W[k * F:(k + 1) * F]
        blocks.append(jnp.concatenate(
            [blk[0:1], jnp.zeros((_IP - 1, W.shape[1]), W.dtype), blk[1:]],
            axis=0))
    return jnp.concatenate(blocks, axis=0)


@jax.jit
def _run(x, node_features, Wf, bf, Wp, bp, Wo, bo,
         enc0_Wru, enc0_bru, enc0_Wc, enc0_bc,
         enc1_Wru, enc1_bru, enc1_Wc, enc1_bc,
         dec0_Wru, dec0_bru, dec0_Wc, dec0_bc,
         dec1_Wru, dec1_bru, dec1_Wc, dec1_bc,
         Wproj, bproj):
    bf16 = jnp.bfloat16
    col = lambda v: v.reshape(-1, 1)
    tb = lambda W: W.T.astype(bf16)
    xfm = jnp.transpose(x[..., 0], (1, 0, 2)).reshape(_P, 1, _BN)
    xin = jnp.concatenate(
        [xfm, jnp.zeros((_P, _IP - 1, _BN), jnp.float32)], axis=1
    ).reshape(_P * _IP, _BN)

    out = pl.pallas_call(
        _gts_kernel,
        out_shape=jax.ShapeDtypeStruct((_Q * _B, _N), jnp.float32),
    )(node_features, node_features.T, Wf, Wf.T, bf.reshape(1, -1), col(bf),
      Wp[_H:2 * _H, :], Wp[0:_H, :].T, col(bp), Wo.T, bo.reshape(1, -1),
      xin,
      tb(_pad_narrow(enc0_Wru)), col(enc0_bru),
      tb(_pad_narrow(enc0_Wc)), col(enc0_bc),
      tb(enc1_Wru), col(enc1_bru), tb(enc1_Wc), col(enc1_bc),
      tb(_pad_narrow(dec0_Wru)), col(dec0_bru),
      tb(_pad_narrow(dec0_Wc)), col(dec0_bc),
      tb(dec1_Wru), col(dec1_bru), tb(dec1_Wc), col(dec1_bc),
      Wproj, bproj.reshape(1, 1))
    return jnp.transpose(out.reshape(_Q, _B, _N), (1, 0, 2))


def kernel(x, As, ycl, iteration, node_features, Wf, bf, Wp, bp, Wo, bo,
           enc0_Wru, enc0_bru, enc0_Wc, enc0_bc,
           enc1_Wru, enc1_bru, enc1_Wc, enc1_bc,
           dec0_Wru, dec0_bru, dec0_Wc, dec0_bc,
           dec1_Wru, dec1_bru, dec1_Wc, dec1_bc,
           Wproj, bproj):
    return _run(x, node_features, Wf, bf, Wp, bp, Wo, bo,
                enc0_Wru, enc0_bru, enc0_Wc, enc0_bc,
                enc1_Wru, enc1_bru, enc1_Wc, enc1_bc,
                dec0_Wru, dec0_bru, dec0_Wc, dec0_bc,
                dec1_Wru, dec1_bru, dec1_Wc, dec1_bc,
                Wproj, bproj)
